# baseline (speedup 1.0000x reference)
import jax
import jax.numpy as jnp
from jax import lax
from jax.experimental import pallas as pl
from jax.experimental.pallas import tpu as pltpu
from jax.experimental.pallas import tpu_sc as plsc

_INFO = plsc.get_sparse_core_info()
_NC = _INFO.num_cores
_NS = _INFO.num_subcores
_NW = _NC * _NS
_CHUNK = 128


def _gather_body(nchunk, table_hbm, idx_hbm, out_hbm, idx_v, rows_v, sem):
    wid = lax.axis_index("s") * _NC + lax.axis_index("c")
    pltpu.sync_copy(idx_hbm.at[wid], idx_v)
    copies = []
    for j in range(nchunk):
        copies.append(
            pltpu.async_copy(table_hbm.at[idx_v.at[j]], rows_v.at[j], sem))
    for c in copies:
        c.wait()
    pltpu.sync_copy(rows_v, out_hbm.at[wid])


def kernel(full_x, indices):
    n = indices.shape[0]
    d = full_x.shape[1]
    nchunk = n // (_NW * _CHUNK)
    idx = indices.astype(jnp.int32).reshape(_NW, nchunk, _CHUNK)

    import functools
    body = functools.partial(_gather_body, nchunk)
    out = pl.kernel(
        body,
        out_type=jax.ShapeDtypeStruct((_NW, nchunk, _CHUNK, d), jnp.float32),
        mesh=plsc.VectorSubcoreMesh(core_axis_name="c", subcore_axis_name="s"),
        scratch_types=[
            pltpu.VMEM((nchunk, _CHUNK), jnp.int32),
            pltpu.VMEM((nchunk, _CHUNK, d), jnp.float32),
            pltpu.SemaphoreType.DMA,
        ],
        compiler_params=pltpu.CompilerParams(use_tc_tiling_on_sc=False),
    )(full_x, idx)
    return out.reshape(n, d)


# speedup vs baseline: 1.7330x; 1.7330x over previous
import functools

import jax
import jax.numpy as jnp
from jax import lax
from jax.experimental import pallas as pl
from jax.experimental.pallas import tpu as pltpu
from jax.experimental.pallas import tpu_sc as plsc

_INFO = plsc.get_sparse_core_info()
_NC = _INFO.num_cores
_NS = _INFO.num_subcores
_NW = _NC * _NS
_L = 16


def _body(n_per_w, d, table_hbm, idx_hbm, out_hbm, idx_v, rows_v, sem):
    wid = lax.axis_index("s") * _NC + lax.axis_index("c")
    base = wid * n_per_w
    pltpu.sync_copy(idx_hbm.at[pl.ds(base, n_per_w)], idx_v)

    def grp_body(g, carry):
        ivec = idx_v[pl.ds(g * _L, _L)]
        for lane in range(_L):
            i = ivec[lane]
            r = g * _L + lane
            pltpu.async_copy(
                table_hbm.at[pl.ds(i, 1)], rows_v.at[pl.ds(r, 1)], sem)
        return carry

    lax.fori_loop(0, n_per_w // _L, grp_body, 0)
    pltpu.make_async_copy(table_hbm.at[pl.ds(0, n_per_w)], rows_v, sem).wait()
    pltpu.sync_copy(rows_v, out_hbm.at[wid])


def kernel(full_x, indices):
    n = indices.shape[0]
    d = full_x.shape[1]
    n_per_w = n // _NW
    idx = indices.astype(jnp.int32)

    body = functools.partial(_body, n_per_w, d)
    out = pl.kernel(
        body,
        out_type=jax.ShapeDtypeStruct((_NW, n_per_w, d), jnp.float32),
        mesh=plsc.VectorSubcoreMesh(core_axis_name="c", subcore_axis_name="s"),
        scratch_types=[
            pltpu.VMEM((n_per_w,), jnp.int32),
            pltpu.VMEM((n_per_w, d), jnp.float32),
            pltpu.SemaphoreType.DMA,
        ],
        compiler_params=pltpu.CompilerParams(
            use_tc_tiling_on_sc=True, needs_layout_passes=False),
    )(full_x, idx)
    return out.reshape(n, d)
